# baseline (speedup 1.0000x reference)
import functools

import jax
import jax.numpy as jnp
from jax.experimental import pallas as pl
from jax.experimental.pallas import tpu as pltpu

T = 8192
D_MODEL = 2048
N_EXPERTS = 16
TILE_T = 512


def _routing_body(alpha_ref, hidden_ref, logits_ref, eg_ref, out_ref):
    alpha = alpha_ref[0, 0]
    sim = jax.lax.dot_general(
        hidden_ref[...],
        eg_ref[...],
        dimension_numbers=(((1,), (1,)), ((), ())),
        preferred_element_type=jnp.float32,
    )
    g = logits_ref[...] + alpha * sim

    idx = jax.lax.broadcasted_iota(jnp.int32, g.shape, 1)
    neg_inf = jnp.float32(-jnp.inf)

    m1 = jnp.max(g, axis=1, keepdims=True)
    i1 = jnp.min(jnp.where(g == m1, idx, N_EXPERTS), axis=1, keepdims=True)
    g2 = jnp.where(idx == i1, neg_inf, g)
    m2 = jnp.max(g2, axis=1, keepdims=True)
    i2 = jnp.min(jnp.where(g2 == m2, idx, N_EXPERTS), axis=1, keepdims=True)

    e = jnp.exp(m2 - m1)
    r = 1.0 / (1.0 + e)
    w1 = r
    w2 = e * r

    out_ref[...] = jnp.concatenate(
        [i1.astype(jnp.float32), w1, i2.astype(jnp.float32), w2], axis=1
    )


@jax.jit
def _run(token_hidden, router_logits, expert_ground, alpha):
    alpha_arr = jnp.reshape(alpha.astype(jnp.float32), (1, 1))
    packed = pl.pallas_call(
        _routing_body,
        grid=(T // TILE_T,),
        in_specs=[
            pl.BlockSpec(memory_space=pltpu.SMEM),
            pl.BlockSpec((TILE_T, D_MODEL), lambda i: (i, 0)),
            pl.BlockSpec((TILE_T, N_EXPERTS), lambda i: (i, 0)),
            pl.BlockSpec((N_EXPERTS, D_MODEL), lambda i: (0, 0)),
        ],
        out_specs=pl.BlockSpec((TILE_T, 4), lambda i: (i, 0)),
        out_shape=jax.ShapeDtypeStruct((T, 4), jnp.float32),
    )(alpha_arr, token_hidden, router_logits, expert_ground)
    return packed.reshape(T, 2, 2)


def kernel(token_hidden, router_logits, expert_ground, alpha):
    return _run(token_hidden, router_logits, expert_ground, alpha)


# speedup vs baseline: 1.0931x; 1.0931x over previous
import jax
import jax.numpy as jnp
from jax.experimental import pallas as pl
from jax.experimental.pallas import tpu as pltpu

T = 8192
D_MODEL = 2048
N_EXPERTS = 16
TILE_T = 512


def _ident(n):
    r = jax.lax.broadcasted_iota(jnp.int32, (n, n), 0)
    c = jax.lax.broadcasted_iota(jnp.int32, (n, n), 1)
    return (r == c).astype(jnp.float32)


def _routing_body(alpha_ref, hidden_ref, logits_ref, eg_ref, out_ref):
    alpha = alpha_ref[0, 0]
    dims = (((1,), (1,)), ((), ()))
    sim_t = jax.lax.dot_general(
        eg_ref[...], hidden_ref[...], dims, preferred_element_type=jnp.float32
    )
    logits_t = jax.lax.dot_general(
        _ident(N_EXPERTS), logits_ref[...], dims,
        preferred_element_type=jnp.float32,
    )
    g = logits_t + alpha * sim_t

    idx = jax.lax.broadcasted_iota(jnp.int32, g.shape, 0)
    neg_inf = jnp.float32(-jnp.inf)

    m1 = jnp.max(g, axis=0, keepdims=True)
    i1 = jnp.min(jnp.where(g == m1, idx, N_EXPERTS), axis=0, keepdims=True)
    g2 = jnp.where(idx == i1, neg_inf, g)
    m2 = jnp.max(g2, axis=0, keepdims=True)
    i2 = jnp.min(jnp.where(g2 == m2, idx, N_EXPERTS), axis=0, keepdims=True)

    e = jnp.exp(m2 - m1)
    r = 1.0 / (1.0 + e)
    w1 = r
    w2 = e * r

    packed_t = jnp.concatenate(
        [i1.astype(jnp.float32), w1, i2.astype(jnp.float32), w2], axis=0
    )
    out_ref[...] = jax.lax.dot_general(
        packed_t, _ident(4), (((0,), (0,)), ((), ())),
        preferred_element_type=jnp.float32,
    )


@jax.jit
def _run(token_hidden, router_logits, expert_ground, alpha):
    alpha_arr = jnp.reshape(alpha.astype(jnp.float32), (1, 1))
    packed = pl.pallas_call(
        _routing_body,
        grid=(T // TILE_T,),
        in_specs=[
            pl.BlockSpec(memory_space=pltpu.SMEM),
            pl.BlockSpec((TILE_T, D_MODEL), lambda i: (i, 0)),
            pl.BlockSpec((TILE_T, N_EXPERTS), lambda i: (i, 0)),
            pl.BlockSpec((N_EXPERTS, D_MODEL), lambda i: (0, 0)),
        ],
        out_specs=pl.BlockSpec((TILE_T, 4), lambda i: (i, 0)),
        out_shape=jax.ShapeDtypeStruct((T, 4), jnp.float32),
        compiler_params=pltpu.CompilerParams(
            dimension_semantics=("arbitrary",),
        ),
    )(alpha_arr, token_hidden, router_logits, expert_ground)
    return packed.reshape(T, 2, 2)


def kernel(token_hidden, router_logits, expert_ground, alpha):
    return _run(token_hidden, router_logits, expert_ground, alpha)


# speedup vs baseline: 1.2288x; 1.1241x over previous
import jax
import jax.numpy as jnp
from jax.experimental import pallas as pl
from jax.experimental.pallas import tpu as pltpu

T = 8192
D_MODEL = 2048
N_EXPERTS = 16
TILE_T = 1024


def _ident(n):
    r = jax.lax.broadcasted_iota(jnp.int32, (n, n), 0)
    c = jax.lax.broadcasted_iota(jnp.int32, (n, n), 1)
    return (r == c).astype(jnp.float32)


def _routing_body(alpha_ref, hidden_ref, logits_ref, eg_ref, out_ref):
    alpha = alpha_ref[0, 0]
    dims = (((1,), (1,)), ((), ()))
    sim_t = jax.lax.dot_general(
        eg_ref[...], hidden_ref[...], dims, preferred_element_type=jnp.float32
    )
    logits_t = jax.lax.dot_general(
        _ident(N_EXPERTS), logits_ref[...], dims,
        preferred_element_type=jnp.float32,
    )
    g = logits_t + alpha * sim_t

    idx = jax.lax.broadcasted_iota(jnp.int32, g.shape, 0)
    neg_inf = jnp.float32(-jnp.inf)

    m1 = jnp.max(g, axis=0, keepdims=True)
    i1 = jnp.min(jnp.where(g == m1, idx, N_EXPERTS), axis=0, keepdims=True)
    g2 = jnp.where(idx == i1, neg_inf, g)
    m2 = jnp.max(g2, axis=0, keepdims=True)
    i2 = jnp.min(jnp.where(g2 == m2, idx, N_EXPERTS), axis=0, keepdims=True)

    e = jnp.exp(m2 - m1)
    r = 1.0 / (1.0 + e)
    w1 = r
    w2 = e * r

    packed_t = jnp.concatenate(
        [i1.astype(jnp.float32), w1, i2.astype(jnp.float32), w2], axis=0
    )
    out_ref[...] = jax.lax.dot_general(
        packed_t, _ident(4), (((0,), (0,)), ((), ())),
        preferred_element_type=jnp.float32,
    )


@jax.jit
def _run(token_hidden, router_logits, expert_ground, alpha):
    alpha_arr = jnp.reshape(alpha.astype(jnp.float32), (1, 1))
    packed = pl.pallas_call(
        _routing_body,
        grid=(T // TILE_T,),
        in_specs=[
            pl.BlockSpec(memory_space=pltpu.SMEM),
            pl.BlockSpec((TILE_T, D_MODEL), lambda i: (i, 0)),
            pl.BlockSpec((TILE_T, N_EXPERTS), lambda i: (i, 0)),
            pl.BlockSpec((N_EXPERTS, D_MODEL), lambda i: (0, 0)),
        ],
        out_specs=pl.BlockSpec((TILE_T, 4), lambda i: (i, 0)),
        out_shape=jax.ShapeDtypeStruct((T, 4), jnp.float32),
        compiler_params=pltpu.CompilerParams(
            dimension_semantics=("arbitrary",),
        ),
    )(alpha_arr, token_hidden, router_logits, expert_ground)
    return packed.reshape(T, 2, 2)


def kernel(token_hidden, router_logits, expert_ground, alpha):
    return _run(token_hidden, router_logits, expert_ground, alpha)
